# baseline (speedup 1.0000x reference)
import functools

import jax
import jax.numpy as jnp
from jax import lax
from jax.experimental import pallas as pl
from jax.experimental.pallas import tpu as pltpu
from jax.experimental.pallas import tpu_sc as plsc


def _build_sc_call(F, V, D, K, B):
    P = B * F
    NC = 2
    L = 16
    NW = 32
    assert P % NW == 0
    per_w = P // NW
    BLK = 128
    assert per_w % BLK == 0
    nblk = per_w // BLK
    groups = BLK // L
    R = K + 1
    NR = R * BLK
    mgroups = NR // L
    chunks = NR // BLK

    mesh = plsc.VectorSubcoreMesh(core_axis_name="c", subcore_axis_name="s")

    @functools.partial(
        pl.kernel,
        out_type=jax.ShapeDtypeStruct((P * D,), jnp.float32),
        mesh=mesh,
        scratch_types=[
            pltpu.VMEM((BLK,), jnp.int32),
            pltpu.VMEM((NR,), jnp.int32),
            pltpu.VMEM((K * BLK,), jnp.int32),
            pltpu.VMEM((D, NR), jnp.int32),
            pltpu.VMEM((D, NR), jnp.float32),
            pltpu.VMEM((D, BLK), jnp.int32),
            pltpu.VMEM((D, BLK), jnp.float32),
            pltpu.SemaphoreType.DMA,
        ],
        interpret=False,
    )
    def sc_call(tab_hbm, adj_hbm, x_hbm, out_hbm,
                xblk, rowids, qlist, wlist, words, owlist, obuf, sem):
        wid = lax.axis_index("s") * NC + lax.axis_index("c")
        pair0 = wid * per_w

        @pl.loop(0, nblk)
        def _block(blk):
            pbase = pair0 + blk * BLK
            pltpu.sync_copy(x_hbm.at[pl.ds(pbase, BLK)], xblk)

            lanes = lax.iota(jnp.int32, L)
            fvs = []
            for g in range(groups):
                pv = lanes + (pbase + g * L)
                fv = (pv % F) * V
                fvs.append(fv)
                idxv = fv + xblk[pl.ds(g * L, L)]
                rowids[pl.ds(g * L, L)] = idxv
                for j in range(K):
                    qlist[pl.ds(j * BLK + g * L, L)] = idxv * K + j

            adj_descs = [
                pltpu.async_copy(adj_hbm.at[qlist.at[pl.ds(j * BLK, BLK)]],
                                 rowids.at[pl.ds((j + 1) * BLK, BLK)], sem)
                for j in range(K)
            ]
            for d_ in adj_descs:
                d_.wait()

            for j in range(K):
                for g in range(groups):
                    sl = pl.ds((j + 1) * BLK + g * L, L)
                    rowids[sl] = rowids[sl] + fvs[g]

            @pl.loop(0, mgroups)
            def _wl(m):
                rb = rowids[pl.ds(m * L, L)] * D
                for d in range(D):
                    wlist[d, pl.ds(m * L, L)] = rb + d

            descs = []
            for d in range(D):
                for c in range(chunks):
                    sl = pl.ds(c * BLK, BLK)
                    descs.append(pltpu.async_copy(
                        tab_hbm.at[wlist.at[d, sl]],
                        words.at[d, sl], sem))
            for d_ in descs:
                d_.wait()

            lanes16 = lanes * D
            for g in range(groups):
                pg = lanes16 + ((pbase + g * L) * D)
                for d in range(D):
                    owlist[d, pl.ds(g * L, L)] = pg + d

            inv_k = jnp.float32(1.0 / K)

            @pl.loop(0, D * groups)
            def _acc(u):
                d = u // groups
                g = u % groups
                sl = pl.ds(g * L, L)

                def term(j):
                    return words[d, pl.ds(j * BLK + g * L, L)]

                acc = term(1)
                for j in range(2, R):
                    acc = acc + term(j)
                obuf[d, sl] = acc * inv_k + term(0)

            out_descs = [
                pltpu.async_copy(obuf.at[d], out_hbm.at[owlist.at[d]], sem)
                for d in range(D)
            ]
            for d_ in out_descs:
                d_.wait()

    return sc_call


def kernel(tables, adj, X):
    F, V, D = tables.shape
    K = adj.shape[-1]
    B = X.shape[0]
    sc_call = _build_sc_call(F, V, D, K, B)
    tab_flat = tables.reshape(F * V * D)
    adj_flat = adj.reshape(F * V * K).astype(jnp.int32)
    x_flat = X.reshape(-1).astype(jnp.int32)
    out = sc_call(tab_flat, adj_flat, x_flat)
    return out.reshape(B, F, D)


# speedup vs baseline: 1.0028x; 1.0028x over previous
import functools

import jax
import jax.numpy as jnp
from jax import lax
from jax.experimental import pallas as pl
from jax.experimental.pallas import tpu as pltpu
from jax.experimental.pallas import tpu_sc as plsc


def _build_sc_call(F, V, D, K, B):
    P = B * F
    NC = 2
    L = 16
    NW = 32
    assert P % NW == 0
    per_w = P // NW
    BLK = 128
    assert per_w % BLK == 0
    nblk = per_w // BLK
    groups = BLK // L
    R = K + 1
    NR = R * BLK
    mgroups = NR // L
    chunks = NR // BLK

    mesh = plsc.VectorSubcoreMesh(core_axis_name="c", subcore_axis_name="s")

    @functools.partial(
        pl.kernel,
        out_type=jax.ShapeDtypeStruct((P * D,), jnp.float32),
        mesh=mesh,
        scratch_types=[
            pltpu.VMEM((BLK,), jnp.int32),
            pltpu.VMEM((NR,), jnp.int32),
            pltpu.VMEM((K, BLK), jnp.int32),
            pltpu.VMEM((D * NR,), jnp.int32),
            pltpu.VMEM((D * NR,), jnp.float32),
            pltpu.VMEM((D, BLK), jnp.int32),
            pltpu.VMEM((D, BLK), jnp.float32),
            pltpu.SemaphoreType.DMA,
        ],
        interpret=False,
    )
    def sc_call(tab_hbm, adj_hbm, x_hbm, out_hbm,
                xblk, rowids, qlist, wlist, words, owlist, obuf, sem):
        wid = lax.axis_index("s") * NC + lax.axis_index("c")
        pair0 = wid * per_w

        @pl.loop(0, nblk)
        def _block(blk):
            pbase = pair0 + blk * BLK
            pltpu.sync_copy(x_hbm.at[pl.ds(pbase, BLK)], xblk)

            lanes = lax.iota(jnp.int32, L)
            fvs = []
            for g in range(groups):
                pv = lanes + (pbase + g * L)
                fv = (pv % F) * V
                fvs.append(fv)
                idxv = fv + xblk[pl.ds(g * L, L)]
                rowids[pl.ds(g * L, L)] = idxv
                for j in range(K):
                    qlist[j, pl.ds(g * L, L)] = idxv * K + j

            adj_descs = [
                pltpu.async_copy(adj_hbm.at[qlist.at[j]],
                                 rowids.at[pl.ds((j + 1) * BLK, BLK)], sem)
                for j in range(K)
            ]
            for d_ in adj_descs:
                d_.wait()

            for j in range(K):
                for g in range(groups):
                    sl = pl.ds((j + 1) * BLK + g * L, L)
                    rowids[sl] = rowids[sl] + fvs[g]

            @pl.loop(0, mgroups)
            def _wl(m):
                rb = rowids[pl.ds(m * L, L)] * D
                for d in range(D):
                    wlist[pl.ds(d * NR + m * L, L)] = rb + d

            descs = [
                pltpu.async_copy(tab_hbm.at[wlist.at[pl.ds(d * NR, NR)]],
                                 words.at[pl.ds(d * NR, NR)], sem)
                for d in range(D)
            ]
            for d_ in descs:
                d_.wait()

            lanes16 = lanes * D
            for g in range(groups):
                pg = lanes16 + ((pbase + g * L) * D)
                for d in range(D):
                    owlist[d, pl.ds(g * L, L)] = pg + d

            inv_k = jnp.float32(1.0 / K)

            @pl.loop(0, D * groups)
            def _acc(u):
                d = u // groups
                g = u % groups
                sl = pl.ds(g * L, L)

                def term(j):
                    return words[pl.ds(d * NR + j * BLK + g * L, L)]

                acc = term(1)
                for j in range(2, R):
                    acc = acc + term(j)
                obuf[d, sl] = acc * inv_k + term(0)

            out_descs = [
                pltpu.async_copy(obuf.at[d], out_hbm.at[owlist.at[d]], sem)
                for d in range(D)
            ]
            for d_ in out_descs:
                d_.wait()

    return sc_call


def kernel(tables, adj, X):
    F, V, D = tables.shape
    K = adj.shape[-1]
    B = X.shape[0]
    sc_call = _build_sc_call(F, V, D, K, B)
    tab_flat = tables.reshape(F * V * D)
    adj_flat = adj.reshape(F * V * K).astype(jnp.int32)
    x_flat = X.reshape(-1).astype(jnp.int32)
    out = sc_call(tab_flat, adj_flat, x_flat)
    return out.reshape(B, F, D)


# speedup vs baseline: 2.1673x; 2.1612x over previous
import functools

import jax
import jax.numpy as jnp
from jax import lax
from jax.experimental import pallas as pl
from jax.experimental.pallas import tpu as pltpu
from jax.experimental.pallas import tpu_sc as plsc


def _build_sc_call(F, V, D, K, B):
    P = B * F
    NC = 2
    L = 16
    NW = 32
    assert P % NW == 0
    per_w = P // NW
    BLK = 128
    assert per_w % BLK == 0
    nblk = per_w // BLK
    groups = BLK // L
    R = K + 1
    NR = R * BLK

    mesh = plsc.VectorSubcoreMesh(core_axis_name="c", subcore_axis_name="s")

    @functools.partial(
        pl.kernel,
        out_type=jax.ShapeDtypeStruct((P * D,), jnp.float32),
        mesh=mesh,
        compiler_params=pltpu.CompilerParams(use_tc_tiling_on_sc=False),
        scratch_types=[
            pltpu.VMEM((BLK,), jnp.int32),
            pltpu.VMEM((NR,), jnp.int32),
            pltpu.VMEM((K, BLK), jnp.int32),
            pltpu.VMEM((NR, D), jnp.float32),
            pltpu.VMEM((BLK * D,), jnp.float32),
            pltpu.SemaphoreType.DMA,
        ],
        interpret=False,
    )
    def sc_call(tab_hbm, adj_hbm, x_hbm, out_hbm,
                xblk, rowids, qlist, rows, outblk, sem):
        wid = lax.axis_index("s") * NC + lax.axis_index("c")
        pair0 = wid * per_w

        @pl.loop(0, nblk)
        def _block(blk):
            pbase = pair0 + blk * BLK
            pltpu.sync_copy(x_hbm.at[pl.ds(pbase, BLK)], xblk)

            lanes = lax.iota(jnp.int32, L)
            fvs = []
            for g in range(groups):
                pv = lanes + (pbase + g * L)
                fv = (pv % F) * V
                fvs.append(fv)
                idxv = fv + xblk[pl.ds(g * L, L)]
                rowids[pl.ds(g * L, L)] = idxv
                for j in range(K):
                    qlist[j, pl.ds(g * L, L)] = idxv * K + j

            adj_descs = [
                pltpu.async_copy(adj_hbm.at[qlist.at[j]],
                                 rowids.at[pl.ds((j + 1) * BLK, BLK)], sem)
                for j in range(K)
            ]
            for d_ in adj_descs:
                d_.wait()

            for j in range(K):
                for g in range(groups):
                    sl = pl.ds((j + 1) * BLK + g * L, L)
                    rowids[sl] = rowids[sl] + fvs[g]

            pltpu.async_copy(tab_hbm.at[rowids], rows, sem).wait()

            inv_k = jnp.float32(1.0 / K)

            @pl.loop(0, BLK)
            def _acc(i):
                acc = rows[BLK + i]
                for j in range(2, R):
                    acc = acc + rows[j * BLK + i]
                outblk[pl.ds(i * D, D)] = acc * inv_k + rows[i]

            pltpu.sync_copy(outblk, out_hbm.at[pl.ds(pbase * D, BLK * D)])

    return sc_call


def kernel(tables, adj, X):
    F, V, D = tables.shape
    K = adj.shape[-1]
    B = X.shape[0]
    sc_call = _build_sc_call(F, V, D, K, B)
    tab_flat = tables.reshape(F * V, D)
    adj_flat = adj.reshape(F * V * K).astype(jnp.int32)
    x_flat = X.reshape(-1).astype(jnp.int32)
    out = sc_call(tab_flat, adj_flat, x_flat)
    return out.reshape(B, F, D)


# speedup vs baseline: 2.1863x; 1.0088x over previous
import functools

import jax
import jax.numpy as jnp
from jax import lax
from jax.experimental import pallas as pl
from jax.experimental.pallas import tpu as pltpu
from jax.experimental.pallas import tpu_sc as plsc


def _build_sc_call(F, V, D, K, B):
    P = B * F
    NC = 2
    L = 16
    NW = 32
    assert P % NW == 0
    per_w = P // NW
    BLK = 256
    assert per_w % BLK == 0
    nblk = per_w // BLK
    groups = BLK // L
    R = K + 1
    NR = R * BLK

    mesh = plsc.VectorSubcoreMesh(core_axis_name="c", subcore_axis_name="s")

    @functools.partial(
        pl.kernel,
        out_type=jax.ShapeDtypeStruct((P * D,), jnp.float32),
        mesh=mesh,
        compiler_params=pltpu.CompilerParams(use_tc_tiling_on_sc=False),
        scratch_types=[
            pltpu.VMEM((BLK,), jnp.int32),
            pltpu.VMEM((NR,), jnp.int32),
            pltpu.VMEM((K, BLK), jnp.int32),
            pltpu.VMEM((NR, D), jnp.float32),
            pltpu.VMEM((BLK * D,), jnp.float32),
            pltpu.SemaphoreType.DMA,
        ],
        interpret=False,
    )
    def sc_call(tab_hbm, adj_hbm, x_hbm, out_hbm,
                xblk, rowids, qlist, rows, outblk, sem):
        wid = lax.axis_index("s") * NC + lax.axis_index("c")
        pair0 = wid * per_w

        @pl.loop(0, nblk)
        def _block(blk):
            pbase = pair0 + blk * BLK
            pltpu.sync_copy(x_hbm.at[pl.ds(pbase, BLK)], xblk)

            lanes = lax.iota(jnp.int32, L)
            fvs = []
            for g in range(groups):
                pv = lanes + (pbase + g * L)
                fv = (pv % F) * V
                fvs.append(fv)
                idxv = fv + xblk[pl.ds(g * L, L)]
                rowids[pl.ds(g * L, L)] = idxv
                for j in range(K):
                    qlist[j, pl.ds(g * L, L)] = idxv * K + j

            adj_descs = [
                pltpu.async_copy(adj_hbm.at[qlist.at[j]],
                                 rowids.at[pl.ds((j + 1) * BLK, BLK)], sem)
                for j in range(K)
            ]
            for d_ in adj_descs:
                d_.wait()

            for j in range(K):
                for g in range(groups):
                    sl = pl.ds((j + 1) * BLK + g * L, L)
                    rowids[sl] = rowids[sl] + fvs[g]

            pltpu.async_copy(tab_hbm.at[rowids], rows, sem).wait()

            inv_k = jnp.float32(1.0 / K)

            @pl.loop(0, BLK)
            def _acc(i):
                acc = rows[BLK + i]
                for j in range(2, R):
                    acc = acc + rows[j * BLK + i]
                outblk[pl.ds(i * D, D)] = acc * inv_k + rows[i]

            pltpu.sync_copy(outblk, out_hbm.at[pl.ds(pbase * D, BLK * D)])

    return sc_call


def kernel(tables, adj, X):
    F, V, D = tables.shape
    K = adj.shape[-1]
    B = X.shape[0]
    sc_call = _build_sc_call(F, V, D, K, B)
    tab_flat = tables.reshape(F * V, D)
    adj_flat = adj.reshape(F * V * K).astype(jnp.int32)
    x_flat = X.reshape(-1).astype(jnp.int32)
    out = sc_call(tab_flat, adj_flat, x_flat)
    return out.reshape(B, F, D)


# speedup vs baseline: 2.1944x; 1.0037x over previous
import functools

import jax
import jax.numpy as jnp
from jax import lax
from jax.experimental import pallas as pl
from jax.experimental.pallas import tpu as pltpu
from jax.experimental.pallas import tpu_sc as plsc


def _build_sc_call(F, V, D, K, B):
    P = B * F
    NC = 2
    L = 16
    NW = 32
    assert P % NW == 0
    per_w = P // NW
    BLK = 416
    assert per_w % BLK == 0
    nblk = per_w // BLK
    groups = BLK // L
    R = K + 1
    NR = R * BLK

    mesh = plsc.VectorSubcoreMesh(core_axis_name="c", subcore_axis_name="s")

    @functools.partial(
        pl.kernel,
        out_type=jax.ShapeDtypeStruct((P * D,), jnp.float32),
        mesh=mesh,
        compiler_params=pltpu.CompilerParams(use_tc_tiling_on_sc=False),
        scratch_types=[
            pltpu.VMEM((BLK,), jnp.int32),
            pltpu.VMEM((NR,), jnp.int32),
            pltpu.VMEM((K, BLK), jnp.int32),
            pltpu.VMEM((NR, D), jnp.float32),
            pltpu.VMEM((BLK * D,), jnp.float32),
            pltpu.SemaphoreType.DMA,
        ],
        interpret=False,
    )
    def sc_call(tab_hbm, adj_hbm, x_hbm, out_hbm,
                xblk, rowids, qlist, rows, outblk, sem):
        wid = lax.axis_index("s") * NC + lax.axis_index("c")
        pair0 = wid * per_w

        @pl.loop(0, nblk)
        def _block(blk):
            pbase = pair0 + blk * BLK
            pltpu.sync_copy(x_hbm.at[pl.ds(pbase, BLK)], xblk)

            lanes = lax.iota(jnp.int32, L)
            fvs = []
            for g in range(groups):
                pv = lanes + (pbase + g * L)
                fv = (pv % F) * V
                fvs.append(fv)
                idxv = fv + xblk[pl.ds(g * L, L)]
                rowids[pl.ds(g * L, L)] = idxv
                for j in range(K):
                    qlist[j, pl.ds(g * L, L)] = idxv * K + j

            adj_descs = [
                pltpu.async_copy(adj_hbm.at[qlist.at[j]],
                                 rowids.at[pl.ds((j + 1) * BLK, BLK)], sem)
                for j in range(K)
            ]
            for d_ in adj_descs:
                d_.wait()

            for j in range(K):
                for g in range(groups):
                    sl = pl.ds((j + 1) * BLK + g * L, L)
                    rowids[sl] = rowids[sl] + fvs[g]

            pltpu.async_copy(tab_hbm.at[rowids], rows, sem).wait()

            inv_k = jnp.float32(1.0 / K)

            @pl.loop(0, BLK)
            def _acc(i):
                acc = rows[BLK + i]
                for j in range(2, R):
                    acc = acc + rows[j * BLK + i]
                outblk[pl.ds(i * D, D)] = acc * inv_k + rows[i]

            pltpu.sync_copy(outblk, out_hbm.at[pl.ds(pbase * D, BLK * D)])

    return sc_call


def kernel(tables, adj, X):
    F, V, D = tables.shape
    K = adj.shape[-1]
    B = X.shape[0]
    sc_call = _build_sc_call(F, V, D, K, B)
    tab_flat = tables.reshape(F * V, D)
    adj_flat = adj.reshape(F * V * K).astype(jnp.int32)
    x_flat = X.reshape(-1).astype(jnp.int32)
    out = sc_call(tab_flat, adj_flat, x_flat)
    return out.reshape(B, F, D)
